# baseline (speedup 1.0000x reference)
import jax
import jax.numpy as jnp
from jax import lax
from jax.experimental import pallas as pl
from jax.experimental.pallas import tpu as pltpu
from jax.experimental.pallas import tpu_sc as plsc

_N = 10000
_E = 320000
_G = 64
_NB = 16
_BN = 632
_NP = _NB * _BN
_RPT = _NP // 16
_CH = 128
_NC0 = 80
_NC = 160
_EPAD = 32 * _NC0 * _CH
_ECH = _EPAD // _CH


def _sc_mesh():
    return plsc.VectorSubcoreMesh(core_axis_name="c", subcore_axis_name="s",
                                  num_cores=2, num_subcores=16)


def _agg_pipeline(h_ref, idx_ref, dst_ref, accum, bufs, ebase, dbase, nc):
    sidx0, didx0, rows0, sem0 = bufs

    def step(i, carry):
        pltpu.sync_copy(idx_ref.at[pl.ds(ebase + i * _CH, _CH)], sidx0)
        pltpu.sync_copy(dst_ref.at[pl.ds(dbase + i * _CH, _CH)], didx0)
        pltpu.async_copy(h_ref.at[sidx0], rows0, sem0).wait()
        pltpu.sync_copy(rows0, accum.at[didx0], add=True)
        return carry

    lax.fori_loop(0, nc, step, 0)


def _sc_scratch():
    return [
        pltpu.VMEM((_CH,), jnp.int32),
        pltpu.VMEM((_CH,), jnp.int32),
        pltpu.VMEM((_CH, 128), jnp.float32),
        pltpu.VMEM_SHARED((_NP, 128), jnp.float32),
        pltpu.SemaphoreType.DMA,
    ]


def _sc_agg0(xp, idx, dstpad, zeros):
    def body(x_ref, idx_ref, dst_ref, zeros_ref, z_ref,
             sidx, didx, rows, accum, sem):
        c = lax.axis_index("c")
        s = lax.axis_index("s")
        ebase = (c * 16 + s) * _NC0 * _CH
        pltpu.sync_copy(zeros_ref, accum.at[pl.ds(s * _RPT, _RPT)])
        plsc.subcore_barrier()
        _agg_pipeline(x_ref, idx_ref, dst_ref, accum,
                      (sidx, didx, rows, sem), ebase, ebase, _NC0)
        plsc.subcore_barrier()
        pltpu.sync_copy(accum.at[pl.ds(s * _RPT, _RPT)],
                        z_ref.at[pl.ds(c * _NP + s * _RPT, _RPT)])

    f = pl.kernel(
        body,
        out_type=jax.ShapeDtypeStruct((2 * _NP, 128), jnp.float32),
        mesh=_sc_mesh(),
        scratch_types=_sc_scratch(),
    )
    return f(xp, idx, dstpad, zeros)


def _sc_agg(hcat, idxcat, dstpad):
    def body(hcat_ref, idx_ref, dst_ref, z_ref,
             sidx, didx, rows, accum, sem):
        c = lax.axis_index("c")
        s = lax.axis_index("s")
        pltpu.sync_copy(hcat_ref.at[pl.ds(c * _NP + s * _RPT, _RPT)],
                        accum.at[pl.ds(s * _RPT, _RPT)])
        plsc.subcore_barrier()
        _agg_pipeline(hcat_ref, idx_ref, dst_ref, accum,
                      (sidx, didx, rows, sem),
                      c * _EPAD + s * _NC * _CH, s * _NC * _CH, _NC)
        plsc.subcore_barrier()
        pltpu.sync_copy(accum.at[pl.ds(s * _RPT, _RPT)],
                        z_ref.at[pl.ds(c * _NP + s * _RPT, _RPT)])

    f = pl.kernel(
        body,
        out_type=jax.ShapeDtypeStruct((2 * _NP, 128), jnp.float32),
        mesh=_sc_mesh(),
        scratch_types=_sc_scratch(),
    )
    return f(hcat, idxcat, dstpad)


def _mlp0(xp, z2, W1, b1, W2, b2):
    def body(x_ref, z_ref, w1_ref, b1_ref, w2_ref, b2_ref, o_ref):
        zz = x_ref[...] + z_ref[0] + z_ref[1]
        t = jnp.maximum(
            jnp.dot(zz, w1_ref[...], preferred_element_type=jnp.float32)
            + b1_ref[...], 0.0)
        h = jnp.dot(t, w2_ref[...], preferred_element_type=jnp.float32) + b2_ref[...]
        h = jnp.maximum(h, 0.0)
        o_ref[0, :, :] = h[:, :128]
        o_ref[1, :, :] = h[:, 128:]

    return pl.pallas_call(
        body,
        grid=(_NB,),
        in_specs=[
            pl.BlockSpec((_BN, 128), lambda i: (i, 0)),
            pl.BlockSpec((2, _BN, 128), lambda i: (0, i, 0)),
            pl.BlockSpec((128, 256), lambda i: (0, 0)),
            pl.BlockSpec((1, 256), lambda i: (0, 0)),
            pl.BlockSpec((256, 256), lambda i: (0, 0)),
            pl.BlockSpec((1, 256), lambda i: (0, 0)),
        ],
        out_specs=pl.BlockSpec((2, _BN, 128), lambda i: (0, i, 0)),
        out_shape=jax.ShapeDtypeStruct((2, _NP, 128), jnp.float32),
    )(xp, z2, W1, b1, W2, b2)


def _mlp(z2, W1r, b1, W2, b2):
    def body(z_ref, w1_ref, b1_ref, w2_ref, b2_ref, o_ref):
        t = (jnp.dot(z_ref[0], w1_ref[0], preferred_element_type=jnp.float32)
             + jnp.dot(z_ref[1], w1_ref[1], preferred_element_type=jnp.float32))
        t = jnp.maximum(t + b1_ref[...], 0.0)
        h = jnp.dot(t, w2_ref[...], preferred_element_type=jnp.float32) + b2_ref[...]
        h = jnp.maximum(h, 0.0)
        o_ref[0, :, :] = h[:, :128]
        o_ref[1, :, :] = h[:, 128:]

    return pl.pallas_call(
        body,
        grid=(_NB,),
        in_specs=[
            pl.BlockSpec((2, _BN, 128), lambda i: (0, i, 0)),
            pl.BlockSpec((2, 128, 256), lambda i: (0, 0, 0)),
            pl.BlockSpec((1, 256), lambda i: (0, 0)),
            pl.BlockSpec((256, 256), lambda i: (0, 0)),
            pl.BlockSpec((1, 256), lambda i: (0, 0)),
        ],
        out_specs=pl.BlockSpec((2, _BN, 128), lambda i: (0, i, 0)),
        out_shape=jax.ShapeDtypeStruct((2, _NP, 128), jnp.float32),
    )(z2, W1r, b1, W2, b2)


def _mlp_pool(z2, W1r, b1, W2, b2, batch3, Wfc, bfc):
    def body(z_ref, w1_ref, b1_ref, w2_ref, b2_ref, b_ref, wfc_ref, bfc_ref,
             o_ref, sums, counts):
        i = pl.program_id(0)

        @pl.when(i == 0)
        def _():
            sums[...] = jnp.zeros_like(sums)
            counts[...] = jnp.zeros_like(counts)

        t = (jnp.dot(z_ref[0], w1_ref[0], preferred_element_type=jnp.float32)
             + jnp.dot(z_ref[1], w1_ref[1], preferred_element_type=jnp.float32))
        t = jnp.maximum(t + b1_ref[...], 0.0)
        h = jnp.dot(t, w2_ref[...], preferred_element_type=jnp.float32) + b2_ref[...]
        h = jnp.maximum(h, 0.0)

        b = b_ref[0, 0, :]
        gids = lax.broadcasted_iota(jnp.int32, (_G, _BN), 0)
        A = (b[None, :] == gids).astype(jnp.float32)
        sums[...] += jnp.dot(A, h, preferred_element_type=jnp.float32)
        counts[...] += jnp.broadcast_to(
            jnp.sum(A, axis=1, keepdims=True), (_G, 256))

        @pl.when(i == _NB - 1)
        def _():
            pooled = sums[...] / jnp.maximum(counts[...], 1.0)
            o_ref[...] = (jnp.dot(pooled, wfc_ref[...],
                                  preferred_element_type=jnp.float32)
                          + bfc_ref[...])

    return pl.pallas_call(
        body,
        grid=(_NB,),
        in_specs=[
            pl.BlockSpec((2, _BN, 128), lambda i: (0, i, 0)),
            pl.BlockSpec((2, 128, 256), lambda i: (0, 0, 0)),
            pl.BlockSpec((1, 256), lambda i: (0, 0)),
            pl.BlockSpec((256, 256), lambda i: (0, 0)),
            pl.BlockSpec((1, 256), lambda i: (0, 0)),
            pl.BlockSpec((1, 1, _BN), lambda i: (i, 0, 0)),
            pl.BlockSpec((256, 128), lambda i: (0, 0)),
            pl.BlockSpec((1, 128), lambda i: (0, 0)),
        ],
        out_specs=pl.BlockSpec((_G, 128), lambda i: (0, 0)),
        out_shape=jax.ShapeDtypeStruct((_G, 128), jnp.float32),
        scratch_shapes=[
            pltpu.VMEM((_G, 256), jnp.float32),
            pltpu.VMEM((_G, 256), jnp.float32),
        ],
    )(z2, W1r, b1, W2, b2, batch3, Wfc, bfc)


def kernel(x, edge_index, edge_attr, batch,
           W1_0, b1_0, W2_0, b2_0,
           W1_1, b1_1, W2_1, b2_1,
           W1_2, b1_2, W2_2, b2_2,
           W_fc, b_fc):
    src = edge_index[0]
    dst = edge_index[1]
    pe = _EPAD - _E
    srcp = jnp.concatenate([src, jnp.zeros((pe,), jnp.int32)])
    idxcat = jnp.concatenate([srcp, srcp + _NP])
    dstpad = jnp.concatenate([dst, jnp.full((pe,), _N, jnp.int32)])

    xp = jnp.pad(x, ((0, _NP - _N), (0, 0)))
    zeros = jnp.zeros((_RPT, 128), jnp.float32)

    z = _sc_agg0(xp, srcp, dstpad, zeros)
    h = _mlp0(xp, z.reshape(2, _NP, 128), W1_0, b1_0.reshape(1, 256),
              W2_0, b2_0.reshape(1, 256))
    z = _sc_agg(h.reshape(2 * _NP, 128), idxcat, dstpad)
    h = _mlp(z.reshape(2, _NP, 128), W1_1.reshape(2, 128, 256),
             b1_1.reshape(1, 256), W2_1, b2_1.reshape(1, 256))
    z = _sc_agg(h.reshape(2 * _NP, 128), idxcat, dstpad)

    batch3 = jnp.concatenate(
        [batch, jnp.full((_NP - _N,), _G, jnp.int32)]).reshape(_NB, 1, _BN)
    return _mlp_pool(z.reshape(2, _NP, 128), W1_2.reshape(2, 128, 256),
                     b1_2.reshape(1, 256), W2_2, b2_2.reshape(1, 256),
                     batch3, W_fc, b_fc.reshape(1, 128))


# speedup vs baseline: 1.4025x; 1.4025x over previous
import jax
import jax.numpy as jnp
from jax import lax
from jax.experimental import pallas as pl
from jax.experimental.pallas import tpu as pltpu
from jax.experimental.pallas import tpu_sc as plsc

_N = 10000
_E = 320000
_G = 64
_NB = 16
_BN = 632
_NP = _NB * _BN
_RPT = _NP // 16
_CH = 128
_NC0 = 79
_NC = 158
_EPAD = 32 * _NC0 * _CH
_ECH = _EPAD // _CH


def _sc_mesh():
    return plsc.VectorSubcoreMesh(core_axis_name="c", subcore_axis_name="s",
                                  num_cores=2, num_subcores=16)


def _agg_pipeline(h_ref, idx_ref, dst_ref, accum, bufs, ebase, dbase, nc):
    sidx0, didx0, rows0, sem0 = bufs

    def step(i, carry):
        pltpu.sync_copy(idx_ref.at[pl.ds(ebase + i * _CH, _CH)], sidx0)
        pltpu.sync_copy(dst_ref.at[pl.ds(dbase + i * _CH, _CH)], didx0)
        pltpu.async_copy(h_ref.at[sidx0], rows0, sem0).wait()
        pltpu.sync_copy(rows0, accum.at[didx0], add=True)
        return carry

    lax.fori_loop(0, nc, step, 0)


def _sc_scratch():
    return [
        pltpu.VMEM((_CH,), jnp.int32),
        pltpu.VMEM((_CH,), jnp.int32),
        pltpu.VMEM((_CH, 128), jnp.float32),
        pltpu.VMEM_SHARED((_NP, 128), jnp.float32),
        pltpu.SemaphoreType.DMA,
    ]


def _sc_agg0(xp, idx, dstpad, zeros):
    def body(x_ref, idx_ref, dst_ref, zeros_ref, z_ref,
             sidx, didx, rows, accum, sem):
        c = lax.axis_index("c")
        s = lax.axis_index("s")
        ebase = (c * 16 + s) * _NC0 * _CH
        pltpu.sync_copy(zeros_ref, accum.at[pl.ds(s * _RPT, _RPT)])
        plsc.subcore_barrier()
        _agg_pipeline(x_ref, idx_ref, dst_ref, accum,
                      (sidx, didx, rows, sem), ebase, ebase, _NC0)
        plsc.subcore_barrier()
        pltpu.sync_copy(accum.at[pl.ds(s * _RPT, _RPT)],
                        z_ref.at[pl.ds(c * _NP + s * _RPT, _RPT)])

    f = pl.kernel(
        body,
        out_type=jax.ShapeDtypeStruct((2 * _NP, 128), jnp.float32),
        mesh=_sc_mesh(),
        scratch_types=_sc_scratch(),
    )
    return f(xp, idx, dstpad, zeros)


def _sc_agg(hcat, idxcat, dstpad):
    def body(hcat_ref, idx_ref, dst_ref, z_ref,
             sidx, didx, rows, accum, sem):
        c = lax.axis_index("c")
        s = lax.axis_index("s")
        pltpu.sync_copy(hcat_ref.at[pl.ds(c * _NP + s * _RPT, _RPT)],
                        accum.at[pl.ds(s * _RPT, _RPT)])
        plsc.subcore_barrier()
        _agg_pipeline(hcat_ref, idx_ref, dst_ref, accum,
                      (sidx, didx, rows, sem),
                      c * _EPAD + s * _NC * _CH, s * _NC * _CH, _NC)
        plsc.subcore_barrier()
        pltpu.sync_copy(accum.at[pl.ds(s * _RPT, _RPT)],
                        z_ref.at[pl.ds(c * _NP + s * _RPT, _RPT)])

    f = pl.kernel(
        body,
        out_type=jax.ShapeDtypeStruct((2 * _NP, 128), jnp.float32),
        mesh=_sc_mesh(),
        scratch_types=_sc_scratch(),
    )
    return f(hcat, idxcat, dstpad)


def _mlp0(xp, z2, W1, b1, W2, b2):
    def body(x_ref, z_ref, w1_ref, b1_ref, w2_ref, b2_ref, o_ref):
        zz = x_ref[...] + z_ref[0] + z_ref[1]
        t = jnp.maximum(
            jnp.dot(zz, w1_ref[...], preferred_element_type=jnp.float32)
            + b1_ref[...], 0.0)
        h = jnp.dot(t, w2_ref[...], preferred_element_type=jnp.float32) + b2_ref[...]
        h = jnp.maximum(h, 0.0)
        o_ref[0, :, :] = h[:, :128]
        o_ref[1, :, :] = h[:, 128:]

    return pl.pallas_call(
        body,
        grid=(_NB,),
        in_specs=[
            pl.BlockSpec((_BN, 128), lambda i: (i, 0)),
            pl.BlockSpec((2, _BN, 128), lambda i: (0, i, 0)),
            pl.BlockSpec((128, 256), lambda i: (0, 0)),
            pl.BlockSpec((1, 256), lambda i: (0, 0)),
            pl.BlockSpec((256, 256), lambda i: (0, 0)),
            pl.BlockSpec((1, 256), lambda i: (0, 0)),
        ],
        out_specs=pl.BlockSpec((2, _BN, 128), lambda i: (0, i, 0)),
        out_shape=jax.ShapeDtypeStruct((2, _NP, 128), jnp.float32),
    )(xp, z2, W1, b1, W2, b2)


def _mlp(z2, W1r, b1, W2, b2):
    def body(z_ref, w1_ref, b1_ref, w2_ref, b2_ref, o_ref):
        t = (jnp.dot(z_ref[0], w1_ref[0], preferred_element_type=jnp.float32)
             + jnp.dot(z_ref[1], w1_ref[1], preferred_element_type=jnp.float32))
        t = jnp.maximum(t + b1_ref[...], 0.0)
        h = jnp.dot(t, w2_ref[...], preferred_element_type=jnp.float32) + b2_ref[...]
        h = jnp.maximum(h, 0.0)
        o_ref[0, :, :] = h[:, :128]
        o_ref[1, :, :] = h[:, 128:]

    return pl.pallas_call(
        body,
        grid=(_NB,),
        in_specs=[
            pl.BlockSpec((2, _BN, 128), lambda i: (0, i, 0)),
            pl.BlockSpec((2, 128, 256), lambda i: (0, 0, 0)),
            pl.BlockSpec((1, 256), lambda i: (0, 0)),
            pl.BlockSpec((256, 256), lambda i: (0, 0)),
            pl.BlockSpec((1, 256), lambda i: (0, 0)),
        ],
        out_specs=pl.BlockSpec((2, _BN, 128), lambda i: (0, i, 0)),
        out_shape=jax.ShapeDtypeStruct((2, _NP, 128), jnp.float32),
    )(z2, W1r, b1, W2, b2)


def _mlp_pool(z2, W1r, b1, W2, b2, batch3, Wfc, bfc):
    def body(z_ref, w1_ref, b1_ref, w2_ref, b2_ref, b_ref, wfc_ref, bfc_ref,
             o_ref, sums, counts):
        i = pl.program_id(0)

        @pl.when(i == 0)
        def _():
            sums[...] = jnp.zeros_like(sums)
            counts[...] = jnp.zeros_like(counts)

        t = (jnp.dot(z_ref[0], w1_ref[0], preferred_element_type=jnp.float32)
             + jnp.dot(z_ref[1], w1_ref[1], preferred_element_type=jnp.float32))
        t = jnp.maximum(t + b1_ref[...], 0.0)
        h = jnp.dot(t, w2_ref[...], preferred_element_type=jnp.float32) + b2_ref[...]
        h = jnp.maximum(h, 0.0)

        b = b_ref[0, 0, :]
        gids = lax.broadcasted_iota(jnp.int32, (_G, _BN), 0)
        A = (b[None, :] == gids).astype(jnp.float32)
        sums[...] += jnp.dot(A, h, preferred_element_type=jnp.float32)
        counts[...] += jnp.broadcast_to(
            jnp.sum(A, axis=1, keepdims=True), (_G, 256))

        @pl.when(i == _NB - 1)
        def _():
            pooled = sums[...] / jnp.maximum(counts[...], 1.0)
            o_ref[...] = (jnp.dot(pooled, wfc_ref[...],
                                  preferred_element_type=jnp.float32)
                          + bfc_ref[...])

    return pl.pallas_call(
        body,
        grid=(_NB,),
        in_specs=[
            pl.BlockSpec((2, _BN, 128), lambda i: (0, i, 0)),
            pl.BlockSpec((2, 128, 256), lambda i: (0, 0, 0)),
            pl.BlockSpec((1, 256), lambda i: (0, 0)),
            pl.BlockSpec((256, 256), lambda i: (0, 0)),
            pl.BlockSpec((1, 256), lambda i: (0, 0)),
            pl.BlockSpec((1, 1, _BN), lambda i: (i, 0, 0)),
            pl.BlockSpec((256, 128), lambda i: (0, 0)),
            pl.BlockSpec((1, 128), lambda i: (0, 0)),
        ],
        out_specs=pl.BlockSpec((_G, 128), lambda i: (0, 0)),
        out_shape=jax.ShapeDtypeStruct((_G, 128), jnp.float32),
        scratch_shapes=[
            pltpu.VMEM((_G, 256), jnp.float32),
            pltpu.VMEM((_G, 256), jnp.float32),
        ],
    )(z2, W1r, b1, W2, b2, batch3, Wfc, bfc)


def kernel(x, edge_index, edge_attr, batch,
           W1_0, b1_0, W2_0, b2_0,
           W1_1, b1_1, W2_1, b2_1,
           W1_2, b1_2, W2_2, b2_2,
           W_fc, b_fc):
    src = edge_index[0]
    dst = edge_index[1]
    pe = _EPAD - _E
    srcp = jnp.concatenate([src, jnp.zeros((pe,), jnp.int32)])
    idxcat = jnp.concatenate([srcp, srcp + _NP])
    dstpad = jnp.concatenate([dst, jnp.full((pe,), _N, jnp.int32)])

    xp = jnp.pad(x, ((0, _NP - _N), (0, 0)))
    zeros = jnp.zeros((_RPT, 128), jnp.float32)

    z = _sc_agg0(xp, srcp, dstpad, zeros)
    h = _mlp0(xp, z.reshape(2, _NP, 128), W1_0, b1_0.reshape(1, 256),
              W2_0, b2_0.reshape(1, 256))
    z = _sc_agg(h.reshape(2 * _NP, 128), idxcat, dstpad)
    h = _mlp(z.reshape(2, _NP, 128), W1_1.reshape(2, 128, 256),
             b1_1.reshape(1, 256), W2_1, b2_1.reshape(1, 256))
    z = _sc_agg(h.reshape(2 * _NP, 128), idxcat, dstpad)

    batch3 = jnp.concatenate(
        [batch, jnp.full((_NP - _N,), _G, jnp.int32)]).reshape(_NB, 1, _BN)
    return _mlp_pool(z.reshape(2, _NP, 128), W1_2.reshape(2, 128, 256),
                     b1_2.reshape(1, 256), W2_2, b2_2.reshape(1, 256),
                     batch3, W_fc, b_fc.reshape(1, 128))


# speedup vs baseline: 1.9595x; 1.3972x over previous
import jax
import jax.numpy as jnp
from jax import lax
from jax.experimental import pallas as pl
from jax.experimental.pallas import tpu as pltpu
from jax.experimental.pallas import tpu_sc as plsc

_N = 10000
_E = 320000
_G = 64
_NB = 16
_BN = 632
_NP = _NB * _BN
_RPT = _NP // 16
_CH = 128
_NC0 = 79
_NC = 158
_EPAD = 32 * _NC0 * _CH
_ECH = _EPAD // _CH


def _sc_mesh():
    return plsc.VectorSubcoreMesh(core_axis_name="c", subcore_axis_name="s",
                                  num_cores=2, num_subcores=16)


def _agg_pipeline(h_ref, idx_ref, dst_ref, accum, bufs, ebase, dbase, nc):
    (sidx0, didx0, rows0, sem0), (sidx1, didx1, rows1, sem1) = bufs

    def load_idx(i, sidx, didx):
        pltpu.sync_copy(idx_ref.at[pl.ds(ebase + i * _CH, _CH)], sidx)
        pltpu.sync_copy(dst_ref.at[pl.ds(dbase + i * _CH, _CH)], didx)

    def gstart(sidx, rows, sem):
        pltpu.async_copy(h_ref.at[sidx], rows, sem)

    def gwait(sidx, rows, sem):
        pltpu.make_async_copy(h_ref.at[sidx], rows, sem).wait()

    def scat(didx, rows):
        pltpu.sync_copy(rows, accum.at[didx], add=True)

    load_idx(0, sidx0, didx0)
    gstart(sidx0, rows0, sem0)

    def step(j, carry):
        i0 = 2 * j
        load_idx(i0 + 1, sidx1, didx1)
        gstart(sidx1, rows1, sem1)
        gwait(sidx0, rows0, sem0)
        scat(didx0, rows0)

        @pl.when(j < nc // 2 - 1)
        def _():
            load_idx(i0 + 2, sidx0, didx0)
            gstart(sidx0, rows0, sem0)

        gwait(sidx1, rows1, sem1)
        scat(didx1, rows1)
        return carry

    lax.fori_loop(0, nc // 2, step, 0)
    if nc % 2:
        load_idx(nc - 1, sidx0, didx0)
        pltpu.async_copy(h_ref.at[sidx0], rows0, sem0).wait()
        scat(didx0, rows0)


def _sc_scratch():
    buf = [
        pltpu.VMEM((_CH,), jnp.int32),
        pltpu.VMEM((_CH,), jnp.int32),
        pltpu.VMEM((_CH, 128), jnp.float32),
        pltpu.SemaphoreType.DMA,
    ]
    return buf + buf + [pltpu.VMEM_SHARED((_NP, 128), jnp.float32)]


def _sc_agg0(xp, idx, dstpad, zeros):
    def body(x_ref, idx_ref, dst_ref, zeros_ref, z_ref,
             s0, d0, r0, g0, s1, d1, r1, g1, accum):
        c = lax.axis_index("c")
        s = lax.axis_index("s")
        ebase = (c * 16 + s) * _NC0 * _CH
        pltpu.sync_copy(zeros_ref, accum.at[pl.ds(s * _RPT, _RPT)])
        plsc.subcore_barrier()
        _agg_pipeline(x_ref, idx_ref, dst_ref, accum,
                      ((s0, d0, r0, g0), (s1, d1, r1, g1)),
                      ebase, ebase, _NC0)
        plsc.subcore_barrier()
        pltpu.sync_copy(accum.at[pl.ds(s * _RPT, _RPT)],
                        z_ref.at[pl.ds(c * _NP + s * _RPT, _RPT)])

    f = pl.kernel(
        body,
        out_type=jax.ShapeDtypeStruct((2 * _NP, 128), jnp.float32),
        mesh=_sc_mesh(),
        scratch_types=_sc_scratch(),
    )
    return f(xp, idx, dstpad, zeros)


def _sc_agg(hcat, idxcat, dstpad):
    def body(hcat_ref, idx_ref, dst_ref, z_ref,
             s0, d0, r0, g0, s1, d1, r1, g1, accum):
        c = lax.axis_index("c")
        s = lax.axis_index("s")
        pltpu.sync_copy(hcat_ref.at[pl.ds(c * _NP + s * _RPT, _RPT)],
                        accum.at[pl.ds(s * _RPT, _RPT)])
        plsc.subcore_barrier()
        _agg_pipeline(hcat_ref, idx_ref, dst_ref, accum,
                      ((s0, d0, r0, g0), (s1, d1, r1, g1)),
                      c * _EPAD + s * _NC * _CH, s * _NC * _CH, _NC)
        plsc.subcore_barrier()
        pltpu.sync_copy(accum.at[pl.ds(s * _RPT, _RPT)],
                        z_ref.at[pl.ds(c * _NP + s * _RPT, _RPT)])

    f = pl.kernel(
        body,
        out_type=jax.ShapeDtypeStruct((2 * _NP, 128), jnp.float32),
        mesh=_sc_mesh(),
        scratch_types=_sc_scratch(),
    )
    return f(hcat, idxcat, dstpad)


def _mlp0(xp, z2, W1, b1, W2, b2):
    def body(x_ref, z_ref, w1_ref, b1_ref, w2_ref, b2_ref, o_ref):
        zz = x_ref[...] + z_ref[0] + z_ref[1]
        t = jnp.maximum(
            jnp.dot(zz, w1_ref[...], preferred_element_type=jnp.float32)
            + b1_ref[...], 0.0)
        h = jnp.dot(t, w2_ref[...], preferred_element_type=jnp.float32) + b2_ref[...]
        h = jnp.maximum(h, 0.0)
        o_ref[0, :, :] = h[:, :128]
        o_ref[1, :, :] = h[:, 128:]

    return pl.pallas_call(
        body,
        grid=(_NB,),
        in_specs=[
            pl.BlockSpec((_BN, 128), lambda i: (i, 0)),
            pl.BlockSpec((2, _BN, 128), lambda i: (0, i, 0)),
            pl.BlockSpec((128, 256), lambda i: (0, 0)),
            pl.BlockSpec((1, 256), lambda i: (0, 0)),
            pl.BlockSpec((256, 256), lambda i: (0, 0)),
            pl.BlockSpec((1, 256), lambda i: (0, 0)),
        ],
        out_specs=pl.BlockSpec((2, _BN, 128), lambda i: (0, i, 0)),
        out_shape=jax.ShapeDtypeStruct((2, _NP, 128), jnp.float32),
    )(xp, z2, W1, b1, W2, b2)


def _mlp(z2, W1r, b1, W2, b2):
    def body(z_ref, w1_ref, b1_ref, w2_ref, b2_ref, o_ref):
        t = (jnp.dot(z_ref[0], w1_ref[0], preferred_element_type=jnp.float32)
             + jnp.dot(z_ref[1], w1_ref[1], preferred_element_type=jnp.float32))
        t = jnp.maximum(t + b1_ref[...], 0.0)
        h = jnp.dot(t, w2_ref[...], preferred_element_type=jnp.float32) + b2_ref[...]
        h = jnp.maximum(h, 0.0)
        o_ref[0, :, :] = h[:, :128]
        o_ref[1, :, :] = h[:, 128:]

    return pl.pallas_call(
        body,
        grid=(_NB,),
        in_specs=[
            pl.BlockSpec((2, _BN, 128), lambda i: (0, i, 0)),
            pl.BlockSpec((2, 128, 256), lambda i: (0, 0, 0)),
            pl.BlockSpec((1, 256), lambda i: (0, 0)),
            pl.BlockSpec((256, 256), lambda i: (0, 0)),
            pl.BlockSpec((1, 256), lambda i: (0, 0)),
        ],
        out_specs=pl.BlockSpec((2, _BN, 128), lambda i: (0, i, 0)),
        out_shape=jax.ShapeDtypeStruct((2, _NP, 128), jnp.float32),
    )(z2, W1r, b1, W2, b2)


def _mlp_pool(z2, W1r, b1, W2, b2, batch3, Wfc, bfc):
    def body(z_ref, w1_ref, b1_ref, w2_ref, b2_ref, b_ref, wfc_ref, bfc_ref,
             o_ref, sums, counts):
        i = pl.program_id(0)

        @pl.when(i == 0)
        def _():
            sums[...] = jnp.zeros_like(sums)
            counts[...] = jnp.zeros_like(counts)

        t = (jnp.dot(z_ref[0], w1_ref[0], preferred_element_type=jnp.float32)
             + jnp.dot(z_ref[1], w1_ref[1], preferred_element_type=jnp.float32))
        t = jnp.maximum(t + b1_ref[...], 0.0)
        h = jnp.dot(t, w2_ref[...], preferred_element_type=jnp.float32) + b2_ref[...]
        h = jnp.maximum(h, 0.0)

        b = b_ref[0, 0, :]
        gids = lax.broadcasted_iota(jnp.int32, (_G, _BN), 0)
        A = (b[None, :] == gids).astype(jnp.float32)
        sums[...] += jnp.dot(A, h, preferred_element_type=jnp.float32)
        counts[...] += jnp.broadcast_to(
            jnp.sum(A, axis=1, keepdims=True), (_G, 256))

        @pl.when(i == _NB - 1)
        def _():
            pooled = sums[...] / jnp.maximum(counts[...], 1.0)
            o_ref[...] = (jnp.dot(pooled, wfc_ref[...],
                                  preferred_element_type=jnp.float32)
                          + bfc_ref[...])

    return pl.pallas_call(
        body,
        grid=(_NB,),
        in_specs=[
            pl.BlockSpec((2, _BN, 128), lambda i: (0, i, 0)),
            pl.BlockSpec((2, 128, 256), lambda i: (0, 0, 0)),
            pl.BlockSpec((1, 256), lambda i: (0, 0)),
            pl.BlockSpec((256, 256), lambda i: (0, 0)),
            pl.BlockSpec((1, 256), lambda i: (0, 0)),
            pl.BlockSpec((1, 1, _BN), lambda i: (i, 0, 0)),
            pl.BlockSpec((256, 128), lambda i: (0, 0)),
            pl.BlockSpec((1, 128), lambda i: (0, 0)),
        ],
        out_specs=pl.BlockSpec((_G, 128), lambda i: (0, 0)),
        out_shape=jax.ShapeDtypeStruct((_G, 128), jnp.float32),
        scratch_shapes=[
            pltpu.VMEM((_G, 256), jnp.float32),
            pltpu.VMEM((_G, 256), jnp.float32),
        ],
    )(z2, W1r, b1, W2, b2, batch3, Wfc, bfc)


def kernel(x, edge_index, edge_attr, batch,
           W1_0, b1_0, W2_0, b2_0,
           W1_1, b1_1, W2_1, b2_1,
           W1_2, b1_2, W2_2, b2_2,
           W_fc, b_fc):
    src = edge_index[0]
    dst = edge_index[1]
    pe = _EPAD - _E
    srcp = jnp.concatenate([src, jnp.zeros((pe,), jnp.int32)])
    idxcat = jnp.concatenate([srcp, srcp + _NP])
    dstpad = jnp.concatenate([dst, jnp.full((pe,), _N, jnp.int32)])

    xp = jnp.pad(x, ((0, _NP - _N), (0, 0)))
    zeros = jnp.zeros((_RPT, 128), jnp.float32)

    z = _sc_agg0(xp, srcp, dstpad, zeros)
    h = _mlp0(xp, z.reshape(2, _NP, 128), W1_0, b1_0.reshape(1, 256),
              W2_0, b2_0.reshape(1, 256))
    z = _sc_agg(h.reshape(2 * _NP, 128), idxcat, dstpad)
    h = _mlp(z.reshape(2, _NP, 128), W1_1.reshape(2, 128, 256),
             b1_1.reshape(1, 256), W2_1, b2_1.reshape(1, 256))
    z = _sc_agg(h.reshape(2 * _NP, 128), idxcat, dstpad)

    batch3 = jnp.concatenate(
        [batch, jnp.full((_NP - _N,), _G, jnp.int32)]).reshape(_NB, 1, _BN)
    return _mlp_pool(z.reshape(2, _NP, 128), W1_2.reshape(2, 128, 256),
                     b1_2.reshape(1, 256), W2_2, b2_2.reshape(1, 256),
                     batch3, W_fc, b_fc.reshape(1, 128))
